# Initial kernel scaffold; baseline (speedup 1.0000x reference)
import jax
import jax.numpy as jnp
from jax.experimental import pallas as pl


def kernel(sentence_embeddings, sentence_spans):
    raise NotImplementedError("write your pallas kernel here")


# speedup vs baseline: 3.5535x; 3.5535x over previous
import functools

import jax
import jax.numpy as jnp
from jax import lax
from jax.experimental import pallas as pl
from jax.experimental.pallas import tpu as pltpu
from jax.experimental.pallas import tpu_sc as plsc

_N_TOK = 32768
_D = 128
_N_SPANS = 1024
_L = 16
_NW = 32
_SPW = _N_SPANS // _NW
_NV = _D // _L
_CH = 128

_mesh = plsc.VectorSubcoreMesh(core_axis_name="c", subcore_axis_name="s")


@functools.partial(
    pl.kernel,
    out_type=jax.ShapeDtypeStruct((_N_SPANS, _D), jnp.float32),
    mesh=_mesh,
    scratch_types=[
        pltpu.VMEM((_SPW + _L,), jnp.int32),
        pltpu.VMEM((_CH, _D), jnp.float32),
        pltpu.VMEM((_SPW, _D), jnp.float32),
    ],
)
def _span_sum(emb_hbm, bounds_hbm, out_hbm, bnd_v, buf, out_v):
    wid = lax.axis_index("c") * 16 + lax.axis_index("s")
    base_span = pl.multiple_of(wid * _SPW, 8)
    pltpu.sync_copy(bounds_hbm.at[pl.ds(base_span, _SPW + _L)], bnd_v)

    @pl.loop(0, _SPW)
    def _zero(j):
        z = jnp.zeros((_L,), jnp.float32)
        for v in range(_NV):
            out_v[j, pl.ds(v * _L, _L)] = z

    t0 = bnd_v[pl.ds(0, _L)][0]
    t0a = (t0 // 8) * 8
    t1 = bnd_v[pl.ds(_SPW, _L)][0]
    nch = (t1 - t0a + (_CH - 1)) // _CH

    @pl.loop(0, nch)
    def _chunk(c):
        lo = t0a + c * _CH
        hi = jnp.minimum(lo + _CH, t1)
        dma_start = pl.multiple_of(jnp.minimum(lo, _N_TOK - _CH), 8)
        pltpu.sync_copy(emb_hbm.at[pl.ds(dma_start, _CH)], buf)

        @pl.loop(0, _SPW)
        def _span(j):
            se = bnd_v[pl.ds(j, _L)]
            rlo = jnp.maximum(se[0], lo)
            rhi = jnp.minimum(se[1], hi)

            @pl.when(rhi > rlo)
            def _():
                init = tuple(jnp.zeros((_L,), jnp.float32) for _ in range(_NV))

                @pl.loop(rlo, rhi, init_carry=init)
                def _acc(r, acc):
                    rb = r - dma_start
                    return tuple(
                        a + buf[rb, pl.ds(v * _L, _L)] for v, a in enumerate(acc)
                    )

                for v in range(_NV):
                    plsc.addupdate(out_v.at[j, pl.ds(v * _L, _L)], _acc[v])

    pltpu.sync_copy(out_v, out_hbm.at[pl.ds(base_span, _SPW)])


@jax.jit
def kernel(sentence_embeddings, sentence_spans):
    bounds = jnp.concatenate(
        [
            sentence_spans[:, 0],
            jnp.full((_L,), _N_TOK, dtype=jnp.int32),
        ]
    )
    return _span_sum(sentence_embeddings, bounds)


# speedup vs baseline: 5.1688x; 1.4546x over previous
import functools

import jax
import jax.numpy as jnp
from jax import lax
from jax.experimental import pallas as pl
from jax.experimental.pallas import tpu as pltpu
from jax.experimental.pallas import tpu_sc as plsc

_N_TOK = 32768
_D = 128
_N_SPANS = 1024
_L = 16
_NW = 32
_SPW = _N_SPANS // _NW
_NV = _D // _L
_CH = 256

_mesh = plsc.VectorSubcoreMesh(core_axis_name="c", subcore_axis_name="s")


@functools.partial(
    pl.kernel,
    out_type=jax.ShapeDtypeStruct((_N_SPANS, _D), jnp.float32),
    mesh=_mesh,
    scratch_types=[
        pltpu.VMEM((_SPW + _L,), jnp.int32),
        pltpu.VMEM((_CH, _D), jnp.float32),
        pltpu.VMEM((_CH, _D), jnp.float32),
        pltpu.VMEM((_SPW, _D), jnp.float32),
        pltpu.SemaphoreType.DMA,
        pltpu.SemaphoreType.DMA,
    ],
)
def _span_sum(emb_hbm, bounds_hbm, out_hbm, bnd_v, buf_a, buf_b, out_v,
              sem_a, sem_b):
    wid = lax.axis_index("c") * 16 + lax.axis_index("s")
    base_span = pl.multiple_of(wid * _SPW, 8)
    pltpu.sync_copy(bounds_hbm.at[pl.ds(base_span, _SPW + _L)], bnd_v)

    @pl.loop(0, _SPW)
    def _zero(j):
        z = jnp.zeros((_L,), jnp.float32)
        for v in range(_NV):
            out_v[j, pl.ds(v * _L, _L)] = z

    t0 = bnd_v[pl.ds(0, _L)][0]
    t0a = (t0 // 8) * 8
    t1 = bnd_v[pl.ds(_SPW, _L)][0]
    nch = (t1 - t0a + (_CH - 1)) // _CH

    def _dma_start(c):
        lo = t0a + c * _CH
        return pl.multiple_of(jnp.minimum(lo, _N_TOK - _CH), 8)

    def _start(c, buf, sem):
        pltpu.async_copy(emb_hbm.at[pl.ds(_dma_start(c), _CH)], buf, sem)

    def _wait(buf, sem):
        pltpu.make_async_copy(emb_hbm.at[pl.ds(0, _CH)], buf, sem).wait()

    def _process(c, buf):
        lo = t0a + c * _CH
        hi = jnp.minimum(lo + _CH, t1)
        dma_start = _dma_start(c)

        @pl.loop(0, _SPW)
        def _span(j):
            se = bnd_v[pl.ds(j, _L)]
            rlo = jnp.maximum(se[0], lo)
            rhi = jnp.minimum(se[1], hi)

            @pl.when(rhi > rlo)
            def _():
                init = tuple(jnp.zeros((_L,), jnp.float32) for _ in range(_NV))

                @pl.loop(rlo, rhi, init_carry=init)
                def _acc(r, acc):
                    rb = r - dma_start
                    return tuple(
                        a + buf[rb, pl.ds(v * _L, _L)] for v, a in enumerate(acc)
                    )

                for v in range(_NV):
                    plsc.addupdate(out_v.at[j, pl.ds(v * _L, _L)], _acc[v])

    @pl.when(nch > 0)
    def _():
        _start(0, buf_a, sem_a)

    @pl.when(nch > 1)
    def _():
        _start(1, buf_b, sem_b)

    @pl.loop(0, (nch + 1) // 2)
    def _grp(g):
        c0 = 2 * g
        _wait(buf_a, sem_a)
        _process(c0, buf_a)

        @pl.when(c0 + 2 < nch)
        def _():
            _start(c0 + 2, buf_a, sem_a)

        @pl.when(c0 + 1 < nch)
        def _():
            _wait(buf_b, sem_b)
            _process(c0 + 1, buf_b)

            @pl.when(c0 + 3 < nch)
            def _():
                _start(c0 + 3, buf_b, sem_b)

    pltpu.sync_copy(out_v, out_hbm.at[pl.ds(base_span, _SPW)])


@jax.jit
def kernel(sentence_embeddings, sentence_spans):
    bounds = jnp.concatenate(
        [
            sentence_spans[:, 0],
            jnp.full((_L,), _N_TOK, dtype=jnp.int32),
        ]
    )
    return _span_sum(sentence_embeddings, bounds)
